# baseline (speedup 1.0000x reference)
import jax
import jax.numpy as jnp
from jax.experimental import pallas as pl

SEQ_LEN = 8192


def _gate_body(gs_ref, out0_ref, out1_ref):
    g0 = gs_ref[0:8, :]
    g1 = gs_ref[8:16, :]
    m = jnp.maximum(g0, g1)
    e0 = jnp.exp(g0 - m)
    e1 = jnp.exp(g1 - m)
    denom = e0 + e1
    s0 = e0 / denom
    s1 = e1 / denom
    sel = g0 >= g1
    out0_ref[...] = jnp.where(sel, 1.0, 0.0) - s0 + s0
    out1_ref[...] = jnp.where(sel, 0.0, 1.0) - s1 + s1


def kernel(x, W, global_gate_score, smoothing_factor):
    del x, W, smoothing_factor
    gt = global_gate_score.T.reshape(16, 1024)
    out0, out1 = pl.pallas_call(
        _gate_body,
        out_shape=(
            jax.ShapeDtypeStruct((8, 1024), jnp.float32),
            jax.ShapeDtypeStruct((8, 1024), jnp.float32),
        ),
    )(gt)
    return out0.reshape(SEQ_LEN), out1.reshape(SEQ_LEN)


# speedup vs baseline: 2.4591x; 2.4591x over previous
import jax
import jax.numpy as jnp
from jax.experimental import pallas as pl

SEQ_LEN = 8192


def _gate_body(gs_ref, out0_ref, out1_ref):
    g0 = gs_ref[0:1, :]
    g1 = gs_ref[1:2, :]
    sel = g0 >= g1
    out0_ref[...] = jnp.where(sel, 1.0, 0.0)
    out1_ref[...] = jnp.where(sel, 0.0, 1.0)


def kernel(x, W, global_gate_score, smoothing_factor):
    del x, W, smoothing_factor
    gt = global_gate_score.T
    out0, out1 = pl.pallas_call(
        _gate_body,
        out_shape=(
            jax.ShapeDtypeStruct((1, SEQ_LEN), jnp.float32),
            jax.ShapeDtypeStruct((1, SEQ_LEN), jnp.float32),
        ),
    )(gt)
    return out0.reshape(SEQ_LEN), out1.reshape(SEQ_LEN)
